# baseline (speedup 1.0000x reference)
import functools

import jax
import jax.numpy as jnp
from jax import lax
from jax.experimental import pallas as pl
from jax.experimental.pallas import tpu as pltpu
from jax.experimental.pallas import tpu_sc as plsc

_NC = 2
_NS = 16
_NW = _NC * _NS
_LANE = 16


@functools.lru_cache(maxsize=None)
def _make_gather(b: int, l: int, v: int, d: int):
    nvb = d // 8
    nk = 1024 // 128
    nlast = nvb - 16 * (nk - 1)
    bb = b // _NW
    n_units = l * nk
    mesh = plsc.VectorSubcoreMesh(core_axis_name="c", subcore_axis_name="s")

    @functools.partial(
        pl.kernel,
        mesh=mesh,
        out_type=jax.ShapeDtypeStruct((l, nvb, _NW, 1024), jnp.float32),
        scratch_types=[
            pltpu.VMEM((l, bb), jnp.int32),
            pltpu.VMEM((bb,), jnp.int32),
            pltpu.VMEM((bb,), jnp.int32),
            pltpu.VMEM((bb,), jnp.int32),
            pltpu.VMEM((bb,), jnp.int32),
            pltpu.VMEM((bb, 128), jnp.float32),
            pltpu.VMEM((bb, 128), jnp.float32),
            pltpu.VMEM((bb, 128), jnp.float32),
            pltpu.VMEM((bb, 128), jnp.float32),
            pltpu.VMEM((16, 1024), jnp.float32),
            pltpu.VMEM((16, 1024), jnp.float32),
            pltpu.SemaphoreType.DMA,
            pltpu.SemaphoreType.DMA,
            pltpu.SemaphoreType.DMA,
            pltpu.SemaphoreType.DMA,
            pltpu.SemaphoreType.DMA,
            pltpu.SemaphoreType.DMA,
        ],
        compiler_params=pltpu.CompilerParams(
            use_tc_tiling_on_sc=False, needs_layout_passes=False
        ),
    )
    def gather_kernel(tableb_hbm, idx_hbm, out_hbm,
                      idx_v, i0, i1, i2, i3, g0, g1, g2, g3, t0, t1,
                      gs0, gs1, gs2, gs3, ss0, ss1):
        w = lax.axis_index("s") * _NC + lax.axis_index("c")
        irefs = (i0, i1, i2, i3)
        grefs = (g0, g1, g2, g3)
        gsems = (gs0, gs1, gs2, gs3)
        trefs = (t0, t1)
        ssems = (ss0, ss1)
        iot = lax.iota(jnp.int32, _LANE)
        ttvecs = [2 * mm + (iot >> 3) for mm in range(8)]
        cb = (iot & 7) << 7

        pltpu.sync_copy(idx_hbm.at[w], idx_v)

        def prep_and_fire(u, q):
            li = u // nk
            k = u % nk
            shift = k * v
            for jj in range(8):
                irefs[q][pl.ds(16 * jj, 16)] = (
                    idx_v[li, pl.ds(16 * jj, 16)] + shift
                )
            pltpu.async_copy(tableb_hbm.at[irefs[q]], grefs[q], gsems[q])

        for q in range(4):
            prep_and_fire(q, q)

        def wait_store(t, ss, k2):
            @pl.when(k2 < nk - 1)
            def _():
                pltpu.make_async_copy(
                    t, out_hbm.at[0, pl.ds(0, 16), 0], ss
                ).wait()

            @pl.when(k2 == nk - 1)
            def _():
                pltpu.make_async_copy(
                    t.at[pl.ds(0, nlast)],
                    out_hbm.at[0, pl.ds(0, nlast), 0],
                    ss,
                ).wait()

        def unit(u, q):
            li = u // nk
            k = u % nk
            g, gs = grefs[q], gsems[q]
            t, ss = trefs[q % 2], ssems[q % 2]
            pltpu.make_async_copy(tableb_hbm.at[irefs[q]], g, gs).wait()

            @pl.when(u >= 2)
            def _():
                wait_store(t, ss, (u - 2) % nk)

            @plsc.parallel_loop(0, 128, unroll=8)
            def trans_row(j):
                jsplat = jnp.full((_LANE,), 0, jnp.int32) + j
                col = cb + jsplat
                for mm in range(8):
                    vals = g[j, pl.ds(16 * mm, 16)]
                    plsc.store_scatter(t, [ttvecs[mm], col], vals)

            @pl.when(u + 4 < n_units)
            def _():
                prep_and_fire(u + 4, q)

            @pl.when(k < nk - 1)
            def _():
                pltpu.async_copy(t, out_hbm.at[li, pl.ds(16 * k, 16), w], ss)

            @pl.when(k == nk - 1)
            def _():
                pltpu.async_copy(
                    t.at[pl.ds(0, nlast)],
                    out_hbm.at[li, pl.ds(16 * (nk - 1), nlast), w],
                    ss,
                )

        def body(p, carry):
            for q in range(4):
                unit(4 * p + q, q)
            return carry

        lax.fori_loop(0, n_units // 4, body, 0)

        wait_store(trefs[0], ssems[0], jnp.int32((n_units - 2) % nk))
        wait_store(trefs[1], ssems[1], jnp.int32((n_units - 1) % nk))

    return gather_kernel


def kernel(x, prob_table):
    b, l = x.shape
    v, d = prob_table.shape
    nvb = d // 8
    table_pad = jnp.pad(prob_table, ((0, 0), (0, 1024 - d)))
    tableb = table_pad.reshape(v, 8, 128).transpose(1, 0, 2).reshape(8 * v, 128)
    idx = x.reshape(_NW, b // _NW, l).transpose(0, 2, 1).astype(jnp.int32)
    out4 = _make_gather(b, l, v, d)(tableb, idx)
    out5 = out4.reshape(l, nvb, _NW, 8, 128)
    return out5.transpose((2, 4, 0, 1, 3)).reshape(b, l, d)


# speedup vs baseline: 1.0247x; 1.0247x over previous
import functools

import jax
import jax.numpy as jnp
from jax import lax
from jax.experimental import pallas as pl
from jax.experimental.pallas import tpu as pltpu
from jax.experimental.pallas import tpu_sc as plsc

_NC = 2
_NS = 16
_NW = _NC * _NS
_LANE = 16


@functools.lru_cache(maxsize=None)
def _make_gather(b: int, l: int, v: int, d: int):
    nvb = d // 8
    nk = 1024 // 128
    nlast = nvb - 16 * (nk - 1)
    bb = b // _NW
    n_units = l * nk
    mesh = plsc.VectorSubcoreMesh(core_axis_name="c", subcore_axis_name="s")

    @functools.partial(
        pl.kernel,
        mesh=mesh,
        out_type=jax.ShapeDtypeStruct((l, nvb, _NW, 1024), jnp.float32),
        scratch_types=[
            pltpu.VMEM((l, bb), jnp.int32),
            pltpu.VMEM((bb,), jnp.int32),
            pltpu.VMEM((bb,), jnp.int32),
            pltpu.VMEM((bb,), jnp.int32),
            pltpu.VMEM((bb,), jnp.int32),
            pltpu.VMEM((bb, 128), jnp.float32),
            pltpu.VMEM((bb, 128), jnp.float32),
            pltpu.VMEM((bb, 128), jnp.float32),
            pltpu.VMEM((bb, 128), jnp.float32),
            pltpu.VMEM((16, 1024), jnp.float32),
            pltpu.VMEM((16, 1024), jnp.float32),
            pltpu.SemaphoreType.DMA,
            pltpu.SemaphoreType.DMA,
            pltpu.SemaphoreType.DMA,
            pltpu.SemaphoreType.DMA,
            pltpu.SemaphoreType.DMA,
            pltpu.SemaphoreType.DMA,
        ],
        compiler_params=pltpu.CompilerParams(
            use_tc_tiling_on_sc=False, needs_layout_passes=False
        ),
    )
    def gather_kernel(tableb_hbm, idx_hbm, out_hbm,
                      idx_v, i0, i1, i2, i3, g0, g1, g2, g3, t0, t1,
                      gs0, gs1, gs2, gs3, ss0, ss1):
        w = lax.axis_index("s") * _NC + lax.axis_index("c")
        irefs = (i0, i1, i2, i3)
        grefs = (g0, g1, g2, g3)
        gsems = (gs0, gs1, gs2, gs3)
        trefs = (t0, t1)
        ssems = (ss0, ss1)
        iot = lax.iota(jnp.int32, _LANE)
        ttvecs = [2 * mm + (iot >> 3) for mm in range(8)]
        cb = (iot & 7) << 7

        pltpu.sync_copy(idx_hbm.at[w], idx_v)

        def prep_and_fire(u, q):
            li = u // nk
            k = u % nk
            shift = k * v
            for jj in range(8):
                irefs[q][pl.ds(16 * jj, 16)] = (
                    idx_v[li, pl.ds(16 * jj, 16)] + shift
                )
            pltpu.async_copy(tableb_hbm.at[irefs[q]], grefs[q], gsems[q])

        for q in range(4):
            prep_and_fire(q, q)

        def wait_store(t, ss, k2):
            @pl.when(k2 < nk - 1)
            def _():
                pltpu.make_async_copy(
                    t, out_hbm.at[0, pl.ds(0, 16), 0], ss
                ).wait()

            @pl.when(k2 == nk - 1)
            def _():
                pltpu.make_async_copy(
                    t.at[pl.ds(0, nlast)],
                    out_hbm.at[0, pl.ds(0, nlast), 0],
                    ss,
                ).wait()

        def unit(u, q):
            li = u // nk
            k = u % nk
            g, gs = grefs[q], gsems[q]
            t, ss = trefs[q % 2], ssems[q % 2]
            pltpu.make_async_copy(tableb_hbm.at[irefs[q]], g, gs).wait()

            @pl.when(u >= 2)
            def _():
                wait_store(t, ss, (u - 2) % nk)

            @plsc.parallel_loop(0, 128, unroll=2)
            def trans_row(j):
                jsplat = jnp.full((_LANE,), 0, jnp.int32) + j
                col = cb + jsplat
                for mm in range(8):
                    vals = g[j, pl.ds(16 * mm, 16)]
                    plsc.store_scatter(t, [ttvecs[mm], col], vals)

            @pl.when(u + 4 < n_units)
            def _():
                prep_and_fire(u + 4, q)

            @pl.when(k < nk - 1)
            def _():
                pltpu.async_copy(t, out_hbm.at[li, pl.ds(16 * k, 16), w], ss)

            @pl.when(k == nk - 1)
            def _():
                pltpu.async_copy(
                    t.at[pl.ds(0, nlast)],
                    out_hbm.at[li, pl.ds(16 * (nk - 1), nlast), w],
                    ss,
                )

        def body(p, carry):
            for q in range(4):
                unit(4 * p + q, q)
            return carry

        lax.fori_loop(0, n_units // 4, body, 0)

        wait_store(trefs[0], ssems[0], jnp.int32((n_units - 2) % nk))
        wait_store(trefs[1], ssems[1], jnp.int32((n_units - 1) % nk))

    return gather_kernel


def kernel(x, prob_table):
    b, l = x.shape
    v, d = prob_table.shape
    nvb = d // 8
    table_pad = jnp.pad(prob_table, ((0, 0), (0, 1024 - d)))
    tableb = table_pad.reshape(v, 8, 128).transpose(1, 0, 2).reshape(8 * v, 128)
    idx = x.reshape(_NW, b // _NW, l).transpose(0, 2, 1).astype(jnp.int32)
    out4 = _make_gather(b, l, v, d)(tableb, idx)
    out5 = out4.reshape(l, nvb, _NW, 8, 128)
    return out5.transpose((2, 4, 0, 1, 3)).reshape(b, l, d)
